# baseline (speedup 1.0000x reference)
import functools

import jax
import jax.numpy as jnp
from jax import lax
from jax.experimental import pallas as pl
from jax.experimental.pallas import tpu as pltpu
from jax.experimental.pallas import tpu_sc as plsc

N = 10000
E = 160000
IN_FEAT = 3703
H1 = 500
H2 = 100
NCLS = 6

NCORE = 2
NSUB = 16
NW = NCORE * NSUB
LANES = 16

B = 64
NBT = 2560
EP = NBT * B
NB_VALID = E // B
NP = 10240
NPS = NP // NSUB
W = 8

_f32 = jnp.float32
_i32 = jnp.int32


def _zero_vmem_2d(ref, rows, cols):
    zv = jnp.zeros((LANES,), _f32)

    def body(r, _):
        for j in range(cols // LANES):
            ref[r, pl.ds(j * LANES, LANES)] = zv
        return 0

    lax.fori_loop(0, rows, body, 0)


def _zero_vmem_1d(ref, n):
    zv = jnp.zeros((LANES,), _f32)

    def body(k, _):
        ref[pl.ds(k * LANES, LANES)] = zv
        return 0

    lax.fori_loop(0, n // LANES, body, 0)


def _sc_deg(row2d):
    mesh = plsc.VectorSubcoreMesh(core_axis_name="c", subcore_axis_name="s")
    nb = NBT // NW

    def body(row_h, d0_h, d1_h, deg_sh, row_v, dg_v, one_v):
        c = lax.axis_index("c")
        s = lax.axis_index("s")
        sb = pl.multiple_of(s * NPS, 8)
        bbase = (c * NSUB + s) * nb

        _zero_vmem_1d(dg_v, NPS)
        pltpu.sync_copy(dg_v, deg_sh.at[pl.ds(sb, NPS)])
        pltpu.sync_copy(row_h.at[pl.ds(bbase, nb)], row_v)
        ov = jnp.ones((LANES,), _f32)
        for j in range(B // LANES):
            one_v[pl.ds(j * LANES, LANES)] = ov
        plsc.subcore_barrier()

        def deg_body(j, _):
            @pl.when(bbase + j < NB_VALID)
            def _():
                pltpu.sync_copy(one_v, deg_sh.at[row_v.at[j]], add=True)
            return 0

        lax.fori_loop(0, nb, deg_body, 0)
        plsc.subcore_barrier()

        @pl.when(c == 0)
        def _():
            pltpu.sync_copy(deg_sh.at[pl.ds(sb, NPS)], d0_h.at[pl.ds(sb, NPS)])

        @pl.when(c == 1)
        def _():
            pltpu.sync_copy(deg_sh.at[pl.ds(sb, NPS)], d1_h.at[pl.ds(sb, NPS)])

    fn = pl.kernel(
        body,
        out_type=(
            jax.ShapeDtypeStruct((NP,), _f32),
            jax.ShapeDtypeStruct((NP,), _f32),
        ),
        mesh=mesh,
        scratch_types=[
            pltpu.VMEM_SHARED((NP,), _f32),
            pltpu.VMEM((nb, B), _i32),
            pltpu.VMEM((NPS,), _f32),
            pltpu.VMEM((B,), _f32),
        ],
        compiler_params=pltpu.CompilerParams(needs_layout_passes=False),
        name="gcn_sc_deg",
    )
    return fn(row2d)


def _tc_dinv(d0, d1):
    def body(a_ref, b_ref, dinv_ref):
        deg = a_ref[...] + b_ref[...] + 1.0
        dinv_ref[...] = lax.rsqrt(deg)

    blk = pl.BlockSpec((NP // 128, 128), lambda: (0, 0))
    return pl.pallas_call(
        body,
        in_specs=[blk, blk],
        out_specs=blk,
        out_shape=jax.ShapeDtypeStruct((NP // 128, 128), _f32),
    )(d0.reshape(NP // 128, 128), d1.reshape(NP // 128, 128))


def _sc_agg(xls, row2d, col2d, chunked):
    nc = len(xls)
    C = 128
    mesh = plsc.VectorSubcoreMesh(core_axis_name="c", subcore_axis_name="s")
    nb = (NBT // NSUB) if chunked else (NBT // NW)
    nwin = nb // W
    nout = nc if chunked else 2
    nzc = NPS // B

    NBUF = 4

    def body(*refs):
        xl_refs = refs[:nc]
        row_h, col_h = refs[nc:nc + 2]
        out_refs = refs[nc + 2:nc + 2 + nout]
        rest = refs[nc + 2 + nout:]
        out_sh, row_v, col_v = rest[:3]
        gs = rest[3:3 + NBUF]
        gsems = rest[3 + NBUF:3 + 2 * NBUF]
        ssems = rest[3 + 2 * NBUF:3 + 3 * NBUF]
        c = lax.axis_index("c")
        s = lax.axis_index("s")
        sb = pl.multiple_of(s * NPS, 8)
        if chunked:
            bbase = s * nb
        else:
            bbase = (c * NSUB + s) * nb

        bufs = tuple(zip(gs, gsems, ssems))

        def run_chunk(xl, flush):
            _zero_vmem_2d(gs[0], B, C)
            for z in range(nzc):
                pltpu.sync_copy(gs[0], out_sh.at[pl.ds(sb + z * B, B)])
            plsc.subcore_barrier()

            def win_body(win, _):
                wbase = pl.multiple_of(bbase + win * W, 8)
                pltpu.sync_copy(row_h.at[pl.ds(wbase, W)], row_v)
                pltpu.sync_copy(col_h.at[pl.ds(wbase, W)], col_v)

                for p, (g, gsem, _ssem) in enumerate(bufs):
                    pltpu.async_copy(xl.at[col_v.at[p]], g, gsem)

                def bb_body(bb, _):
                    for p, (g, gsem, ssem) in enumerate(bufs):
                        bidx = bb * NBUF + p
                        pltpu.make_async_copy(
                            xl.at[col_v.at[bidx]], g, gsem).wait()
                        pltpu.async_copy(
                            g, out_sh.at[row_v.at[bidx]], ssem, add=True)

                        @pl.when(bidx + NBUF < W)
                        def _():
                            pltpu.make_async_copy(
                                g, out_sh.at[row_v.at[bidx]], ssem).wait()
                            pltpu.async_copy(
                                xl.at[col_v.at[bidx + NBUF]], g, gsem)
                    return 0

                lax.fori_loop(0, W // NBUF, bb_body, 0)
                for p, (g, _gsem, ssem) in enumerate(bufs):
                    pltpu.make_async_copy(
                        g, out_sh.at[row_v.at[W - NBUF + p]], ssem).wait()
                return 0

            lax.fori_loop(0, nwin, win_body, 0)
            plsc.subcore_barrier()
            flush()

        if chunked:
            for ci in range(nc):
                @pl.when(c == (ci & 1))
                def _(ci=ci):
                    def flush(ci=ci):
                        pltpu.sync_copy(out_sh.at[pl.ds(sb, NPS)],
                                        out_refs[ci].at[pl.ds(sb, NPS)])
                    run_chunk(xl_refs[ci], flush)
        else:
            def flush():
                @pl.when(c == 0)
                def _():
                    pltpu.sync_copy(out_sh.at[pl.ds(sb, NPS)],
                                    out_refs[0].at[pl.ds(sb, NPS)])

                @pl.when(c == 1)
                def _():
                    pltpu.sync_copy(out_sh.at[pl.ds(sb, NPS)],
                                    out_refs[1].at[pl.ds(sb, NPS)])
            run_chunk(xl_refs[0], flush)

    fn = pl.kernel(
        body,
        out_type=tuple(jax.ShapeDtypeStruct((NP, C), _f32)
                       for _ in range(nout)),
        mesh=mesh,
        scratch_types=(
            [
                pltpu.VMEM_SHARED((NP, C), _f32),
                pltpu.VMEM((W, B), _i32),
                pltpu.VMEM((W, B), _i32),
            ]
            + [pltpu.VMEM((B, C), _f32)] * 4
            + [pltpu.SemaphoreType.DMA] * 8
        ),
        compiler_params=pltpu.CompilerParams(needs_layout_passes=False),
        name="gcn_sc_agg",
    )
    return fn(*xls, row2d, col2d)


BM = 256
GM = NP // BM
NC1 = 4


def _row_mask(m):
    rowid = m * BM + lax.broadcasted_iota(_i32, (BM, 1), 0)
    return rowid < N


def _tc_mm1(x, w1p, b1p, dinv):
    def body(*refs):
        x_ref, w_ref, b_ref, dv_ref = refs[:4]
        outs = refs[4:]
        m = pl.program_id(0)
        res = jnp.dot(x_ref[...], w_ref[...],
                      preferred_element_type=_f32) + b_ref[...]
        res = jnp.where(_row_mask(m), dv_ref[...] * res, 0.0)
        for t in range(NC1):
            outs[t][...] = res[:, t * 128:(t + 1) * 128]

    return pl.pallas_call(
        body,
        grid=(GM,),
        in_specs=[
            pl.BlockSpec((BM, IN_FEAT), lambda m: (m, 0)),
            pl.BlockSpec((IN_FEAT, 512), lambda m: (0, 0)),
            pl.BlockSpec((1, 512), lambda m: (0, 0)),
            pl.BlockSpec((BM, 1), lambda m: (m, 0)),
        ],
        out_specs=[pl.BlockSpec((BM, 128), lambda m: (m, 0))] * NC1,
        out_shape=[jax.ShapeDtypeStruct((NP, 128), _f32)] * NC1,
    )(x, w1p, b1p, dinv)


def _tc_mm2(a1s, t1s, dinv, w2s, b2p):
    def body(*refs):
        a = refs[0:NC1]
        t = refs[NC1:2 * NC1]
        dv_ref = refs[2 * NC1]
        w = refs[2 * NC1 + 1:2 * NC1 + 1 + NC1]
        b_ref = refs[2 * NC1 + 1 + NC1]
        out = refs[2 * NC1 + 2 + NC1]
        m = pl.program_id(0)
        acc = jnp.broadcast_to(b_ref[...], (BM, 128)).astype(_f32)
        dsv = dv_ref[...]
        for k in range(NC1):
            h = jnp.tanh(dsv * (a[k][...] + t[k][...]))
            acc = acc + jnp.dot(h, w[k][...], preferred_element_type=_f32)
        out[...] = jnp.where(_row_mask(m), dsv * acc, 0.0)

    blk = pl.BlockSpec((BM, 128), lambda m: (m, 0))
    return pl.pallas_call(
        body,
        grid=(GM,),
        in_specs=(
            [blk] * NC1 + [blk] * NC1
            + [pl.BlockSpec((BM, 1), lambda m: (m, 0))]
            + [pl.BlockSpec((128, 128), lambda m: (0, 0))] * NC1
            + [pl.BlockSpec((1, 128), lambda m: (0, 0))]
        ),
        out_specs=blk,
        out_shape=jax.ShapeDtypeStruct((NP, 128), _f32),
    )(*a1s, *t1s, dinv, *w2s, b2p)


def _tc_mm3(p20, p21, t2, dinv, w3p, b3p):
    def body(a_ref, b_ref, t_ref, dv_ref, w_ref, bias_ref, out):
        m = pl.program_id(0)
        dsv = dv_ref[...]
        h = jnp.tanh(dsv * (a_ref[...] + b_ref[...] + t_ref[...]))
        res = jnp.dot(h, w_ref[...],
                      preferred_element_type=_f32) + bias_ref[...]
        out[...] = jnp.where(_row_mask(m), dsv * res, 0.0)

    blk = pl.BlockSpec((BM, 128), lambda m: (m, 0))
    return pl.pallas_call(
        body,
        grid=(GM,),
        in_specs=[
            blk, blk, blk,
            pl.BlockSpec((BM, 1), lambda m: (m, 0)),
            pl.BlockSpec((128, 128), lambda m: (0, 0)),
            pl.BlockSpec((1, 128), lambda m: (0, 0)),
        ],
        out_specs=blk,
        out_shape=jax.ShapeDtypeStruct((NP, 128), _f32),
    )(p20, p21, t2, dinv, w3p, b3p)


def _tc_final(p30, p31, t3, dinv):
    def body(a_ref, b_ref, t_ref, dv_ref, out):
        z = jnp.tanh(dv_ref[...] * (a_ref[...] + b_ref[...] + t_ref[...]))
        colid = lax.broadcasted_iota(_i32, (BM, 128), 1)
        zm = jnp.where(colid < NCLS, z, -1e30)
        m = jnp.max(zm, axis=1, keepdims=True)
        e = jnp.exp(zm - m)
        ssum = jnp.sum(e, axis=1, keepdims=True)
        out[...] = (e / ssum)[:, :NCLS]

    blk = pl.BlockSpec((BM, 128), lambda m: (m, 0))
    return pl.pallas_call(
        body,
        grid=(GM,),
        in_specs=[
            blk, blk, blk,
            pl.BlockSpec((BM, 1), lambda m: (m, 0)),
        ],
        out_specs=pl.BlockSpec((BM, NCLS), lambda m: (m, 0)),
        out_shape=jax.ShapeDtypeStruct((N, NCLS), _f32),
    )(p30, p31, t3, dinv)


def kernel(x, edge_index, W1, b1, W2, b2, W3, b3):
    row = edge_index[0].astype(_i32)
    col = edge_index[1].astype(_i32)
    pad = EP - E
    padv = jnp.full((pad,), N, _i32)
    row2d = jnp.concatenate([row, padv]).reshape(NBT, B)
    col2d = jnp.concatenate([col, padv]).reshape(NBT, B)

    d0, d1 = _sc_deg(row2d)
    dinv2 = _tc_dinv(d0, d1)
    dinv2d = dinv2.reshape(NP)[:, None]

    w1p = jnp.pad(W1, ((0, 0), (0, 512 - H1)))
    b1p = jnp.pad(b1, (0, 512 - H1))[None, :]
    t1s = _tc_mm1(x, w1p, b1p, dinv2d)

    a1s = _sc_agg(t1s, row2d, col2d, chunked=True)

    w2p = jnp.pad(W2, ((0, 512 - H1), (0, 128 - H2)))
    w2s = [w2p[128 * k:128 * (k + 1)] for k in range(NC1)]
    b2p = jnp.pad(b2, (0, 128 - H2))[None, :]
    t2 = _tc_mm2(a1s, t1s, dinv2d, w2s, b2p)

    p20, p21 = _sc_agg([t2], row2d, col2d, chunked=False)

    w3p = jnp.pad(W3, ((0, 128 - H2), (0, 128 - NCLS)))
    b3p = jnp.pad(b3, (0, 128 - NCLS))[None, :]
    t3 = _tc_mm3(p20, p21, t2, dinv2d, w3p, b3p)

    p30, p31 = _sc_agg([t3], row2d, col2d, chunked=False)

    return _tc_final(p30, p31, t3, dinv2d)


# speedup vs baseline: 1.1654x; 1.1654x over previous
import functools

import jax
import jax.numpy as jnp
from jax import lax
from jax.experimental import pallas as pl
from jax.experimental.pallas import tpu as pltpu
from jax.experimental.pallas import tpu_sc as plsc

N = 10000
E = 160000
IN_FEAT = 3703
H1 = 500
H2 = 100
NCLS = 6

NCORE = 2
NSUB = 16
NW = NCORE * NSUB
LANES = 16

B = 128
NBT = 1280
EP = NBT * B
NB_VALID = E // B
NP = 10240
NPS = NP // NSUB
W = 40

_f32 = jnp.float32
_i32 = jnp.int32


def _zero_vmem_2d(ref, rows, cols):
    zv = jnp.zeros((LANES,), _f32)

    def body(r, _):
        for j in range(cols // LANES):
            ref[r, pl.ds(j * LANES, LANES)] = zv
        return 0

    lax.fori_loop(0, rows, body, 0)


def _zero_vmem_1d(ref, n):
    zv = jnp.zeros((LANES,), _f32)

    def body(k, _):
        ref[pl.ds(k * LANES, LANES)] = zv
        return 0

    lax.fori_loop(0, n // LANES, body, 0)


def _sc_deg(row2d):
    mesh = plsc.VectorSubcoreMesh(core_axis_name="c", subcore_axis_name="s")
    nb = NBT // NW

    def body(row_h, d0_h, d1_h, deg_sh, row_v, dg_v, one_v):
        c = lax.axis_index("c")
        s = lax.axis_index("s")
        sb = pl.multiple_of(s * NPS, 8)
        bbase = (c * NSUB + s) * nb

        _zero_vmem_1d(dg_v, NPS)
        pltpu.sync_copy(dg_v, deg_sh.at[pl.ds(sb, NPS)])
        pltpu.sync_copy(row_h.at[pl.ds(bbase, nb)], row_v)
        ov = jnp.ones((LANES,), _f32)
        for j in range(B // LANES):
            one_v[pl.ds(j * LANES, LANES)] = ov
        plsc.subcore_barrier()

        def deg_body(j, _):
            @pl.when(bbase + j < NB_VALID)
            def _():
                pltpu.sync_copy(one_v, deg_sh.at[row_v.at[j]], add=True)
            return 0

        lax.fori_loop(0, nb, deg_body, 0)
        plsc.subcore_barrier()

        @pl.when(c == 0)
        def _():
            pltpu.sync_copy(deg_sh.at[pl.ds(sb, NPS)], d0_h.at[pl.ds(sb, NPS)])

        @pl.when(c == 1)
        def _():
            pltpu.sync_copy(deg_sh.at[pl.ds(sb, NPS)], d1_h.at[pl.ds(sb, NPS)])

    fn = pl.kernel(
        body,
        out_type=(
            jax.ShapeDtypeStruct((NP,), _f32),
            jax.ShapeDtypeStruct((NP,), _f32),
        ),
        mesh=mesh,
        scratch_types=[
            pltpu.VMEM_SHARED((NP,), _f32),
            pltpu.VMEM((nb, B), _i32),
            pltpu.VMEM((NPS,), _f32),
            pltpu.VMEM((B,), _f32),
        ],
        compiler_params=pltpu.CompilerParams(needs_layout_passes=False),
        name="gcn_sc_deg",
    )
    return fn(row2d)


def _tc_dinv(d0, d1):
    def body(a_ref, b_ref, dinv_ref):
        deg = a_ref[...] + b_ref[...] + 1.0
        dinv_ref[...] = lax.rsqrt(deg)

    blk = pl.BlockSpec((NP // 128, 128), lambda: (0, 0))
    return pl.pallas_call(
        body,
        in_specs=[blk, blk],
        out_specs=blk,
        out_shape=jax.ShapeDtypeStruct((NP // 128, 128), _f32),
    )(d0.reshape(NP // 128, 128), d1.reshape(NP // 128, 128))


def _sc_agg(xls, row2d, col2d, chunked):
    nc = len(xls)
    C = 128
    mesh = plsc.VectorSubcoreMesh(core_axis_name="c", subcore_axis_name="s")
    nb = (NBT // NSUB) if chunked else (NBT // NW)
    nwin = nb // W
    nout = nc if chunked else 2
    nzc = NPS // B

    NBUF = 2

    def body(*refs):
        xl_refs = refs[:nc]
        row_h, col_h = refs[nc:nc + 2]
        out_refs = refs[nc + 2:nc + 2 + nout]
        rest = refs[nc + 2 + nout:]
        out_sh, row_v, col_v = rest[:3]
        gs = rest[3:3 + NBUF]
        gsems = rest[3 + NBUF:3 + 2 * NBUF]
        ssems = rest[3 + 2 * NBUF:3 + 3 * NBUF]
        c = lax.axis_index("c")
        s = lax.axis_index("s")
        sb = pl.multiple_of(s * NPS, 8)
        if chunked:
            bbase = s * nb
        else:
            bbase = (c * NSUB + s) * nb

        bufs = tuple(zip(gs, gsems, ssems))

        def run_chunk(xl, flush):
            _zero_vmem_2d(gs[0], B, C)
            for z in range(nzc):
                pltpu.sync_copy(gs[0], out_sh.at[pl.ds(sb + z * B, B)])
            plsc.subcore_barrier()

            def win_body(win, _):
                wbase = pl.multiple_of(bbase + win * W, 8)
                pltpu.sync_copy(row_h.at[pl.ds(wbase, W)], row_v)
                pltpu.sync_copy(col_h.at[pl.ds(wbase, W)], col_v)

                for p, (g, gsem, _ssem) in enumerate(bufs):
                    pltpu.async_copy(xl.at[col_v.at[p]], g, gsem)

                def bb_body(bb, _):
                    for p, (g, gsem, ssem) in enumerate(bufs):
                        bidx = bb * NBUF + p
                        pltpu.make_async_copy(
                            xl.at[col_v.at[bidx]], g, gsem).wait()
                        pltpu.async_copy(
                            g, out_sh.at[row_v.at[bidx]], ssem, add=True)

                        @pl.when(bidx + NBUF < W)
                        def _():
                            pltpu.make_async_copy(
                                g, out_sh.at[row_v.at[bidx]], ssem).wait()
                            pltpu.async_copy(
                                xl.at[col_v.at[bidx + NBUF]], g, gsem)
                    return 0

                lax.fori_loop(0, W // NBUF, bb_body, 0)
                for p, (g, _gsem, ssem) in enumerate(bufs):
                    pltpu.make_async_copy(
                        g, out_sh.at[row_v.at[W - NBUF + p]], ssem).wait()
                return 0

            lax.fori_loop(0, nwin, win_body, 0)
            plsc.subcore_barrier()
            flush()

        if chunked:
            for ci in range(nc):
                @pl.when(c == (ci & 1))
                def _(ci=ci):
                    def flush(ci=ci):
                        pltpu.sync_copy(out_sh.at[pl.ds(sb, NPS)],
                                        out_refs[ci].at[pl.ds(sb, NPS)])
                    run_chunk(xl_refs[ci], flush)
        else:
            def flush():
                @pl.when(c == 0)
                def _():
                    pltpu.sync_copy(out_sh.at[pl.ds(sb, NPS)],
                                    out_refs[0].at[pl.ds(sb, NPS)])

                @pl.when(c == 1)
                def _():
                    pltpu.sync_copy(out_sh.at[pl.ds(sb, NPS)],
                                    out_refs[1].at[pl.ds(sb, NPS)])
            run_chunk(xl_refs[0], flush)

    fn = pl.kernel(
        body,
        out_type=tuple(jax.ShapeDtypeStruct((NP, C), _f32)
                       for _ in range(nout)),
        mesh=mesh,
        scratch_types=(
            [
                pltpu.VMEM_SHARED((NP, C), _f32),
                pltpu.VMEM((W, B), _i32),
                pltpu.VMEM((W, B), _i32),
            ]
            + [pltpu.VMEM((B, C), _f32)] * 2
            + [pltpu.SemaphoreType.DMA] * 4
        ),
        compiler_params=pltpu.CompilerParams(needs_layout_passes=False),
        name="gcn_sc_agg",
    )
    return fn(*xls, row2d, col2d)


BM = 256
GM = NP // BM
NC1 = 4


def _row_mask(m):
    rowid = m * BM + lax.broadcasted_iota(_i32, (BM, 1), 0)
    return rowid < N


def _tc_mm1(x, w1p, b1p, dinv):
    def body(*refs):
        x_ref, w_ref, b_ref, dv_ref = refs[:4]
        outs = refs[4:]
        m = pl.program_id(0)
        res = jnp.dot(x_ref[...], w_ref[...],
                      preferred_element_type=_f32) + b_ref[...]
        res = jnp.where(_row_mask(m), dv_ref[...] * res, 0.0)
        for t in range(NC1):
            outs[t][...] = res[:, t * 128:(t + 1) * 128]

    return pl.pallas_call(
        body,
        grid=(GM,),
        in_specs=[
            pl.BlockSpec((BM, IN_FEAT), lambda m: (m, 0)),
            pl.BlockSpec((IN_FEAT, 512), lambda m: (0, 0)),
            pl.BlockSpec((1, 512), lambda m: (0, 0)),
            pl.BlockSpec((BM, 1), lambda m: (m, 0)),
        ],
        out_specs=[pl.BlockSpec((BM, 128), lambda m: (m, 0))] * NC1,
        out_shape=[jax.ShapeDtypeStruct((NP, 128), _f32)] * NC1,
    )(x, w1p, b1p, dinv)


def _tc_mm2(a1s, t1s, dinv, w2s, b2p):
    def body(*refs):
        a = refs[0:NC1]
        t = refs[NC1:2 * NC1]
        dv_ref = refs[2 * NC1]
        w = refs[2 * NC1 + 1:2 * NC1 + 1 + NC1]
        b_ref = refs[2 * NC1 + 1 + NC1]
        out = refs[2 * NC1 + 2 + NC1]
        m = pl.program_id(0)
        acc = jnp.broadcast_to(b_ref[...], (BM, 128)).astype(_f32)
        dsv = dv_ref[...]
        for k in range(NC1):
            h = jnp.tanh(dsv * (a[k][...] + t[k][...]))
            acc = acc + jnp.dot(h, w[k][...], preferred_element_type=_f32)
        out[...] = jnp.where(_row_mask(m), dsv * acc, 0.0)

    blk = pl.BlockSpec((BM, 128), lambda m: (m, 0))
    return pl.pallas_call(
        body,
        grid=(GM,),
        in_specs=(
            [blk] * NC1 + [blk] * NC1
            + [pl.BlockSpec((BM, 1), lambda m: (m, 0))]
            + [pl.BlockSpec((128, 128), lambda m: (0, 0))] * NC1
            + [pl.BlockSpec((1, 128), lambda m: (0, 0))]
        ),
        out_specs=blk,
        out_shape=jax.ShapeDtypeStruct((NP, 128), _f32),
    )(*a1s, *t1s, dinv, *w2s, b2p)


def _tc_mm3(p20, p21, t2, dinv, w3p, b3p):
    def body(a_ref, b_ref, t_ref, dv_ref, w_ref, bias_ref, out):
        m = pl.program_id(0)
        dsv = dv_ref[...]
        h = jnp.tanh(dsv * (a_ref[...] + b_ref[...] + t_ref[...]))
        res = jnp.dot(h, w_ref[...],
                      preferred_element_type=_f32) + bias_ref[...]
        out[...] = jnp.where(_row_mask(m), dsv * res, 0.0)

    blk = pl.BlockSpec((BM, 128), lambda m: (m, 0))
    return pl.pallas_call(
        body,
        grid=(GM,),
        in_specs=[
            blk, blk, blk,
            pl.BlockSpec((BM, 1), lambda m: (m, 0)),
            pl.BlockSpec((128, 128), lambda m: (0, 0)),
            pl.BlockSpec((1, 128), lambda m: (0, 0)),
        ],
        out_specs=blk,
        out_shape=jax.ShapeDtypeStruct((NP, 128), _f32),
    )(p20, p21, t2, dinv, w3p, b3p)


def _tc_final(p30, p31, t3, dinv):
    def body(a_ref, b_ref, t_ref, dv_ref, out):
        z = jnp.tanh(dv_ref[...] * (a_ref[...] + b_ref[...] + t_ref[...]))
        colid = lax.broadcasted_iota(_i32, (BM, 128), 1)
        zm = jnp.where(colid < NCLS, z, -1e30)
        m = jnp.max(zm, axis=1, keepdims=True)
        e = jnp.exp(zm - m)
        ssum = jnp.sum(e, axis=1, keepdims=True)
        out[...] = (e / ssum)[:, :NCLS]

    blk = pl.BlockSpec((BM, 128), lambda m: (m, 0))
    return pl.pallas_call(
        body,
        grid=(GM,),
        in_specs=[
            blk, blk, blk,
            pl.BlockSpec((BM, 1), lambda m: (m, 0)),
        ],
        out_specs=pl.BlockSpec((BM, NCLS), lambda m: (m, 0)),
        out_shape=jax.ShapeDtypeStruct((N, NCLS), _f32),
    )(p30, p31, t3, dinv)


def kernel(x, edge_index, W1, b1, W2, b2, W3, b3):
    row = edge_index[0].astype(_i32)
    col = edge_index[1].astype(_i32)
    pad = EP - E
    padv = jnp.full((pad,), N, _i32)
    row2d = jnp.concatenate([row, padv]).reshape(NBT, B)
    col2d = jnp.concatenate([col, padv]).reshape(NBT, B)

    d0, d1 = _sc_deg(row2d)
    dinv2 = _tc_dinv(d0, d1)
    dinv2d = dinv2.reshape(NP)[:, None]

    w1p = jnp.pad(W1, ((0, 0), (0, 512 - H1)))
    b1p = jnp.pad(b1, (0, 512 - H1))[None, :]
    t1s = _tc_mm1(x, w1p, b1p, dinv2d)

    a1s = _sc_agg(t1s, row2d, col2d, chunked=True)

    w2p = jnp.pad(W2, ((0, 512 - H1), (0, 128 - H2)))
    w2s = [w2p[128 * k:128 * (k + 1)] for k in range(NC1)]
    b2p = jnp.pad(b2, (0, 128 - H2))[None, :]
    t2 = _tc_mm2(a1s, t1s, dinv2d, w2s, b2p)

    p20, p21 = _sc_agg([t2], row2d, col2d, chunked=False)

    w3p = jnp.pad(W3, ((0, 128 - H2), (0, 128 - NCLS)))
    b3p = jnp.pad(b3, (0, 128 - NCLS))[None, :]
    t3 = _tc_mm3(p20, p21, t2, dinv2d, w3p, b3p)

    p30, p31 = _sc_agg([t3], row2d, col2d, chunked=False)

    return _tc_final(p30, p31, t3, dinv2d)
